# baseline (speedup 1.0000x reference)
import functools

import jax
import jax.numpy as jnp
from jax import lax
from jax.experimental import pallas as pl
from jax.experimental.pallas import tpu as pltpu
from jax.experimental.pallas import tpu_sc as plsc

_EMBED = 64
_BATCH = 16384
_REG_LAMBDA = 0.0001

_NC = 2
_NS = 16
_NW = _NC * _NS
_BPW = _BATCH // _NW
_CH = 128
_NCHUNK = _BPW // _CH
_GROUPS = _BPW // 16


@functools.partial(
    pl.kernel,
    mesh=plsc.VectorSubcoreMesh(core_axis_name="c", subcore_axis_name="s"),
    compiler_params=pltpu.CompilerParams(
        needs_layout_passes=False, use_tc_tiling_on_sc=False),
    out_type=(
        jax.ShapeDtypeStruct((_BATCH,), jnp.float32),
        jax.ShapeDtypeStruct((_NW, 16), jnp.float32),
    ),
    scratch_types=[
        pltpu.VMEM((_NCHUNK, _CH), jnp.int32),
        pltpu.VMEM((_NCHUNK, _CH), jnp.int32),
        pltpu.VMEM((_NCHUNK, _CH), jnp.int32),
        pltpu.VMEM((_BPW, _EMBED), jnp.float32),
        pltpu.VMEM((_BPW, _EMBED), jnp.float32),
        pltpu.VMEM((_BPW, _EMBED), jnp.float32),
        pltpu.VMEM((_BPW,), jnp.float32),
        pltpu.VMEM((16,), jnp.float32),
        pltpu.SemaphoreType.DMA,
    ],
)
def _sc_scores(user_hbm, pos_hbm, neg_hbm, utab_hbm, itab_hbm,
               s_out, reg_out,
               idx_u, idx_p, idx_n, rows_u, rows_p, rows_n,
               s_loc, ssq_loc, sem):
    wid = lax.axis_index("s") * _NC + lax.axis_index("c")
    r0 = wid * _NCHUNK
    pltpu.sync_copy(user_hbm.at[pl.ds(r0, _NCHUNK)], idx_u)
    pltpu.sync_copy(pos_hbm.at[pl.ds(r0, _NCHUNK)], idx_p)
    pltpu.sync_copy(neg_hbm.at[pl.ds(r0, _NCHUNK)], idx_n)

    copies = []
    for k in range(_NCHUNK):
        dst = pl.ds(k * _CH, _CH)
        copies.append(pltpu.async_copy(utab_hbm.at[idx_u.at[k]], rows_u.at[dst], sem))
        copies.append(pltpu.async_copy(itab_hbm.at[idx_p.at[k]], rows_p.at[dst], sem))
        copies.append(pltpu.async_copy(itab_hbm.at[idx_n.at[k]], rows_n.at[dst], sem))
    for c in copies:
        c.wait()

    zero = jnp.zeros((16,), jnp.float32)

    def group(g, sq_carry):
        sq0, sq1 = sq_carry
        rowv = lax.iota(jnp.int32, 16) + g * 16

        def dstep(dv, carry):
            acc0, acc1, q0, q1 = carry
            d0 = dv * 4
            for j in range(4):
                colv = jnp.full((16,), d0 + j, jnp.int32)
                cu = plsc.load_gather(rows_u, [rowv, colv])
                cp = plsc.load_gather(rows_p, [rowv, colv])
                cn = plsc.load_gather(rows_n, [rowv, colv])
                prod = cu * (cp - cn)
                sq = cu * cu + (cp * cp + cn * cn)
                if j % 2 == 0:
                    acc0 = acc0 + prod
                    q0 = q0 + sq
                else:
                    acc1 = acc1 + prod
                    q1 = q1 + sq
            return acc0, acc1, q0, q1

        acc0, acc1, sq0, sq1 = lax.fori_loop(
            0, _EMBED // 4, dstep, (zero, zero, sq0, sq1))
        s_loc[pl.ds(g * 16, 16)] = acc0 + acc1
        return sq0, sq1

    sq0, sq1 = lax.fori_loop(0, _GROUPS, group, (zero, zero))
    ssq_loc[...] = sq0 + sq1
    pltpu.sync_copy(s_loc, s_out.at[pl.ds(wid * _BPW, _BPW)])
    pltpu.sync_copy(ssq_loc, reg_out.at[wid])


def _tc_finish(s_ref, regs_ref, bpr_ref, reg_ref):
    s = s_ref[...]
    sp = jnp.maximum(-s, 0.0) + jnp.log1p(jnp.exp(-jnp.abs(s)))
    bpr_ref[0, 0] = jnp.sum(sp) * (1.0 / _BATCH)
    reg_ref[0, 0] = jnp.sum(regs_ref[...]) * (_REG_LAMBDA / (2.0 * _BATCH))


_tc_fin = pl.pallas_call(
    _tc_finish,
    out_shape=(
        jax.ShapeDtypeStruct((1, 1), jnp.float32),
        jax.ShapeDtypeStruct((1, 1), jnp.float32),
    ),
    in_specs=[
        pl.BlockSpec(memory_space=pltpu.VMEM),
        pl.BlockSpec(memory_space=pltpu.VMEM),
    ],
    out_specs=(
        pl.BlockSpec(memory_space=pltpu.SMEM),
        pl.BlockSpec(memory_space=pltpu.SMEM),
    ),
)


def kernel(user, positive, negative, user_table, item_table):
    u2 = user.reshape(_BATCH // 128, 128)
    p2 = positive.reshape(_BATCH // 128, 128)
    n2 = negative.reshape(_BATCH // 128, 128)
    s, regs = _sc_scores(u2, p2, n2, user_table, item_table)
    bpr, reg = _tc_fin(s.reshape(128, 128), regs.reshape(4, 128))
    return (bpr[0, 0], reg[0, 0])


# speedup vs baseline: 1.5449x; 1.5449x over previous
import functools

import jax
import jax.numpy as jnp
from jax import lax
from jax.experimental import pallas as pl
from jax.experimental.pallas import tpu as pltpu
from jax.experimental.pallas import tpu_sc as plsc

_EMBED = 64
_BATCH = 16384
_REG_LAMBDA = 0.0001

_NC = 2
_NS = 16
_NW = _NC * _NS
_BPW = _BATCH // _NW
_GROUPS = _BPW // 16
_ROW_BYTES = _EMBED * 4
_TABLE_BYTES = _BPW * _ROW_BYTES


@functools.partial(
    pl.kernel,
    mesh=plsc.VectorSubcoreMesh(core_axis_name="c", subcore_axis_name="s"),
    compiler_params=pltpu.CompilerParams(needs_layout_passes=False),
    out_type=(
        jax.ShapeDtypeStruct((128, 128), jnp.float32),
        jax.ShapeDtypeStruct((_NW, 16), jnp.float32),
    ),
    scratch_types=[
        pltpu.VMEM((4, 128), jnp.int32),
        pltpu.VMEM((4, 128), jnp.int32),
        pltpu.VMEM((4, 128), jnp.int32),
        pltpu.VMEM((_BPW // 2, 128), jnp.float32),
        pltpu.VMEM((_BPW // 2, 128), jnp.float32),
        pltpu.VMEM((_BPW // 2, 128), jnp.float32),
        pltpu.VMEM((4, 128), jnp.float32),
        pltpu.VMEM((16,), jnp.float32),
        pltpu.SemaphoreType.DMA,
    ],
)
def _sc_scores(user_hbm, pos_hbm, neg_hbm, utab_hbm, itab_hbm,
               s_out, reg_out,
               idx_u, idx_p, idx_n, rows_u, rows_p, rows_n,
               s_loc, ssq_loc, sem):
    wid = lax.axis_index("s") * _NC + lax.axis_index("c")
    r0 = wid * 4
    pltpu.sync_copy(user_hbm.at[pl.ds(r0, 4)], idx_u)
    pltpu.sync_copy(pos_hbm.at[pl.ds(r0, 4)], idx_p)
    pltpu.sync_copy(neg_hbm.at[pl.ds(r0, 4)], idx_n)

    def fetch(t, _):
        a, b = t >> 3, (t & 7) * 16
        vu = idx_u[a, pl.ds(b, 16)]
        vp = idx_p[a, pl.ds(b, 16)]
        vn = idx_n[a, pl.ds(b, 16)]
        for lane in range(16):
            c = t * 8 + (lane // 2)
            dst = pl.ds((lane & 1) * _EMBED, _EMBED)
            pltpu.make_async_copy(utab_hbm.at[vu[lane]], rows_u.at[c, dst], sem).start()
            pltpu.make_async_copy(itab_hbm.at[vp[lane]], rows_p.at[c, dst], sem).start()
            pltpu.make_async_copy(itab_hbm.at[vn[lane]], rows_n.at[c, dst], sem).start()
        return 0

    lax.fori_loop(0, _BPW // 16, fetch, 0)
    for buf in (rows_u, rows_p, rows_n):
        pltpu.make_async_copy(s_out, buf.at[pl.ds(0, 128)], sem).wait()
        pltpu.make_async_copy(s_out, buf.at[pl.ds(128, 128)], sem).wait()

    zero = jnp.zeros((16,), jnp.float32)

    def group(g, sq_carry):
        sq0, sq1 = sq_carry
        base = (lax.iota(jnp.int32, 16) + g * 16) * _EMBED

        def dstep(dv, carry):
            acc0, acc1, q0, q1 = carry
            d0 = dv * 4
            for j in range(4):
                flat = base + (d0 + j)
                i0 = lax.shift_right_logical(flat, 7)
                i1 = lax.bitwise_and(flat, 127)
                cu = plsc.load_gather(rows_u, [i0, i1])
                cp = plsc.load_gather(rows_p, [i0, i1])
                cn = plsc.load_gather(rows_n, [i0, i1])
                prod = cu * (cp - cn)
                sq = cu * cu + (cp * cp + cn * cn)
                if j % 2 == 0:
                    acc0 = acc0 + prod
                    q0 = q0 + sq
                else:
                    acc1 = acc1 + prod
                    q1 = q1 + sq
            return acc0, acc1, q0, q1

        acc0, acc1, sq0, sq1 = lax.fori_loop(
            0, _EMBED // 4, dstep, (zero, zero, sq0, sq1))
        s_loc[g >> 3, pl.ds((g & 7) * 16, 16)] = acc0 + acc1
        return sq0, sq1

    sq0, sq1 = lax.fori_loop(0, _GROUPS, group, (zero, zero))
    ssq_loc[...] = sq0 + sq1
    pltpu.sync_copy(s_loc, s_out.at[pl.ds(wid * 4, 4)])
    pltpu.sync_copy(ssq_loc, reg_out.at[wid])


def _tc_finish(s_ref, regs_ref, bpr_ref, reg_ref):
    s = s_ref[...]
    sp = jnp.maximum(-s, 0.0) + jnp.log1p(jnp.exp(-jnp.abs(s)))
    bpr_ref[0, 0] = jnp.sum(sp) * (1.0 / _BATCH)
    reg_ref[0, 0] = jnp.sum(regs_ref[...]) * (_REG_LAMBDA / (2.0 * _BATCH))


_tc_fin = pl.pallas_call(
    _tc_finish,
    out_shape=(
        jax.ShapeDtypeStruct((1, 1), jnp.float32),
        jax.ShapeDtypeStruct((1, 1), jnp.float32),
    ),
    in_specs=[
        pl.BlockSpec(memory_space=pltpu.VMEM),
        pl.BlockSpec(memory_space=pltpu.VMEM),
    ],
    out_specs=(
        pl.BlockSpec(memory_space=pltpu.SMEM),
        pl.BlockSpec(memory_space=pltpu.SMEM),
    ),
)


def kernel(user, positive, negative, user_table, item_table):
    u2 = user.reshape(_BATCH // 128, 128)
    p2 = positive.reshape(_BATCH // 128, 128)
    n2 = negative.reshape(_BATCH // 128, 128)
    s, regs = _sc_scores(u2, p2, n2, user_table, item_table)
    bpr, reg = _tc_fin(s, regs.reshape(4, 128))
    return (bpr[0, 0], reg[0, 0])
